# Initial kernel scaffold; baseline (speedup 1.0000x reference)
import jax
import jax.numpy as jnp
from jax.experimental import pallas as pl


def kernel(atom_types, chirals, coordinates, atom_table, chiral_table, W_coord, b_coord):
    raise NotImplementedError("write your pallas kernel here")


# speedup vs baseline: 3.8309x; 3.8309x over previous
import functools

import jax
import jax.numpy as jnp
from jax import lax
from jax.experimental import pallas as pl
from jax.experimental.pallas import tpu as pltpu
from jax.experimental.pallas import tpu_sc as plsc

B, L, D = 1024, 200, 128
N = B * L
NC, NS, LANES = 2, 16, 16
NW = NC * NS
TOK_PER_W = N // NW
T = 128
CHUNKS = TOK_PER_W // T
DC = D // LANES


def _build_comb_table(atom_table, chiral_table, b_coord2d):
  def body(at_ref, ct_ref, b_ref, out_ref):
    cb = ct_ref[...] + b_ref[...]
    out_ref[...] = at_ref[...][:, None, :] + cb[None, :, :]

  return pl.pallas_call(
      body,
      out_shape=jax.ShapeDtypeStruct((1000, 4, D), jnp.float32),
  )(atom_table, chiral_table, b_coord2d)


_mesh = plsc.VectorSubcoreMesh(core_axis_name="c", subcore_axis_name="s")


@functools.partial(
    pl.kernel,
    out_type=jax.ShapeDtypeStruct((N, D), jnp.float32),
    mesh=_mesh,
    scratch_types=[
        pltpu.VMEM((T,), jnp.int32),
        pltpu.VMEM((T,), jnp.int32),
        pltpu.VMEM((T * 3 + LANES,), jnp.float32),
        pltpu.VMEM((T, D), jnp.float32),
        pltpu.VMEM((3 * D,), jnp.float32),
        pltpu.SemaphoreType.DMA,
    ],
)
def _sc_embed(aidx_hbm, cidx_hbm, coords_hbm, comb_hbm, w_hbm, out_hbm,
              idx_v, cidx_v, coords_v, rows_v, w_v, sem):
  wid = lax.axis_index("s") * NC + lax.axis_index("c")
  base_w = wid * TOK_PER_W

  pltpu.sync_copy(w_hbm, w_v)
  w_chunks = [[w_v[pl.ds(r * D + d * LANES, LANES)] for d in range(DC)]
              for r in range(3)]

  def chunk_body(g, carry):
    base = base_w + g * T
    pltpu.sync_copy(aidx_hbm.at[pl.ds(base, T)], idx_v)
    pltpu.sync_copy(cidx_hbm.at[pl.ds(base, T)], cidx_v)
    pltpu.sync_copy(coords_hbm.at[pl.ds(base * 3, T * 3)],
                    coords_v.at[pl.ds(0, T * 3)])
    for i in range(T // LANES):
      s = pl.ds(i * LANES, LANES)
      idx_v[s] = idx_v[s] * 4 + cidx_v[s]
    pltpu.async_copy(comb_hbm.at[idx_v], rows_v, sem).wait()

    def token_body(t, c):
      xyz = coords_v[pl.ds(3 * t, LANES)]
      x = xyz[0]
      y = xyz[1]
      z = xyz[2]
      for d in range(DC):
        s = pl.ds(d * LANES, LANES)
        proj = x * w_chunks[0][d] + y * w_chunks[1][d] + z * w_chunks[2][d]
        rows_v[t, s] = rows_v[t, s] + proj
      return c

    lax.fori_loop(0, T, token_body, 0)
    pltpu.sync_copy(rows_v, out_hbm.at[pl.ds(base, T)])
    return carry

  lax.fori_loop(0, CHUNKS, chunk_body, 0)


def kernel(atom_types, chirals, coordinates, atom_table, chiral_table,
           W_coord, b_coord):
  comb = _build_comb_table(atom_table, chiral_table,
                           jnp.reshape(b_coord, (1, D)))
  comb = jnp.reshape(comb, (4000, D))
  aidx = jnp.reshape(atom_types, (N,)).astype(jnp.int32)
  cidx = jnp.reshape(chirals, (N,)).astype(jnp.int32)
  coords = jnp.reshape(coordinates, (N * 3,))
  w_flat = jnp.reshape(W_coord, (3 * D,))
  out = _sc_embed(aidx, cidx, coords, comb, w_flat)
  return jnp.reshape(out, (B, L, D))
